# baseline (speedup 1.0000x reference)
import functools

import jax
import jax.numpy as jnp
from jax.experimental import pallas as pl
from jax.experimental.pallas import tpu as pltpu

_BLK = 512


def _fused_mlp_kernel(em_ref, x_ref, w1_ref, b1_ref, w2_ref, b2_ref, o_ref,
                      *, hidden, num_experts):
    x = x_ref[...]
    th = (em_ref[...] + 1) * (hidden // num_experts)
    n_chunks = 4
    cw = hidden // n_chunks
    hs = []
    for j in range(n_chunks):
        w1c = w1_ref[pl.ds(j * cw, cw), :]
        z = jax.lax.dot_general(x, w1c, (((1,), (1,)), ((), ())),
                                preferred_element_type=jnp.float32)
        z = z + b1_ref[:, pl.ds(j * cw, cw)]
        col = jax.lax.broadcasted_iota(jnp.int32, z.shape, 1) + j * cw
        h = z * z * jax.nn.sigmoid(z)
        hs.append(jnp.where(col < th, h, 0.0))
    hfull = jnp.concatenate(hs, axis=1)
    out = jax.lax.dot_general(hfull, w2_ref[...], (((1,), (1,)), ((), ())),
                              preferred_element_type=jnp.float32)
    o_ref[...] = out + b2_ref[...]


def kernel(x, expert_mask, W1, b1, W2, b2):
    n_tokens, in_features = x.shape
    hidden = W1.shape[0]
    num_experts = 8
    em2d = expert_mask.reshape(n_tokens, 1)
    b1r = b1.reshape(1, hidden)
    b2r = b2.reshape(1, in_features)
    grid = (n_tokens // _BLK,)
    return pl.pallas_call(
        functools.partial(_fused_mlp_kernel, hidden=hidden,
                          num_experts=num_experts),
        grid=grid,
        in_specs=[
            pl.BlockSpec((_BLK, 1), lambda i: (i, 0)),
            pl.BlockSpec((_BLK, in_features), lambda i: (i, 0)),
            pl.BlockSpec((hidden, in_features), lambda i: (0, 0)),
            pl.BlockSpec((1, hidden), lambda i: (0, 0)),
            pl.BlockSpec((in_features, hidden), lambda i: (0, 0)),
            pl.BlockSpec((1, in_features), lambda i: (0, 0)),
        ],
        out_specs=pl.BlockSpec((_BLK, in_features), lambda i: (i, 0)),
        out_shape=jax.ShapeDtypeStruct((n_tokens, in_features), jnp.float32),
        compiler_params=pltpu.CompilerParams(
            dimension_semantics=("parallel",)),
    )(em2d, x, W1, b1r, W2, b2r)


# speedup vs baseline: 1.1635x; 1.1635x over previous
import functools

import jax
import jax.numpy as jnp
from jax.experimental import pallas as pl
from jax.experimental.pallas import tpu as pltpu

_BLK = 1024


def _fused_mlp_kernel(em_ref, x_ref, w1_ref, b1_ref, w2_ref, b2_ref, o_ref,
                      *, hidden, num_experts):
    x = x_ref[...]
    z = jax.lax.dot_general(x, w1_ref[...], (((1,), (1,)), ((), ())),
                            preferred_element_type=jnp.float32)
    z = z + b1_ref[...]
    th = (em_ref[...] + 1) * (hidden // num_experts)
    col = jax.lax.broadcasted_iota(jnp.int32, z.shape, 1)
    h = z * z * jax.nn.sigmoid(z)
    h = jnp.where(col < th, h, 0.0)
    out = jax.lax.dot_general(h, w2_ref[...], (((1,), (1,)), ((), ())),
                              preferred_element_type=jnp.float32)
    o_ref[...] = out + b2_ref[...]


def kernel(x, expert_mask, W1, b1, W2, b2):
    n_tokens, in_features = x.shape
    hidden = W1.shape[0]
    num_experts = 8
    em2d = expert_mask.reshape(n_tokens, 1)
    b1r = b1.reshape(1, hidden)
    b2r = b2.reshape(1, in_features)
    grid = (n_tokens // _BLK,)
    return pl.pallas_call(
        functools.partial(_fused_mlp_kernel, hidden=hidden,
                          num_experts=num_experts),
        grid=grid,
        in_specs=[
            pl.BlockSpec((_BLK, 1), lambda i: (i, 0)),
            pl.BlockSpec((_BLK, in_features), lambda i: (i, 0)),
            pl.BlockSpec((hidden, in_features), lambda i: (0, 0)),
            pl.BlockSpec((1, hidden), lambda i: (0, 0)),
            pl.BlockSpec((in_features, hidden), lambda i: (0, 0)),
            pl.BlockSpec((1, in_features), lambda i: (0, 0)),
        ],
        out_specs=pl.BlockSpec((_BLK, in_features), lambda i: (i, 0)),
        out_shape=jax.ShapeDtypeStruct((n_tokens, in_features), jnp.float32),
        compiler_params=pltpu.CompilerParams(
            dimension_semantics=("parallel",)),
    )(em2d, x, W1, b1r, W2, b2r)


# speedup vs baseline: 1.1984x; 1.0300x over previous
import functools

import jax
import jax.numpy as jnp
from jax.experimental import pallas as pl
from jax.experimental.pallas import tpu as pltpu

_BLK = 1024


def _fused_mlp_kernel(em_ref, x_ref, w1_ref, b1_ref, w2_ref, b2_ref, o_ref,
                      *, hidden, num_experts):
    x = x_ref[...]
    z = jax.lax.dot_general(x, w1_ref[...], (((1,), (1,)), ((), ())),
                            preferred_element_type=jnp.float32)
    z = z + b1_ref[...]
    em_col = jnp.transpose(em_ref[...], (1, 0))
    th = (em_col + 1) * (hidden // num_experts)
    col = jax.lax.broadcasted_iota(jnp.int32, z.shape, 1)
    h = z * z * jax.nn.sigmoid(z)
    h = jnp.where(col < th, h, 0.0)
    out = jax.lax.dot_general(h, w2_ref[...], (((1,), (1,)), ((), ())),
                              preferred_element_type=jnp.float32)
    o_ref[...] = out + b2_ref[...]


def kernel(x, expert_mask, W1, b1, W2, b2):
    n_tokens, in_features = x.shape
    hidden = W1.shape[0]
    num_experts = 8
    em2d = expert_mask.reshape(1, n_tokens)
    b1r = b1.reshape(1, hidden)
    b2r = b2.reshape(1, in_features)
    grid = (n_tokens // _BLK,)
    return pl.pallas_call(
        functools.partial(_fused_mlp_kernel, hidden=hidden,
                          num_experts=num_experts),
        grid=grid,
        in_specs=[
            pl.BlockSpec((1, _BLK), lambda i: (0, i)),
            pl.BlockSpec((_BLK, in_features), lambda i: (i, 0)),
            pl.BlockSpec((hidden, in_features), lambda i: (0, 0)),
            pl.BlockSpec((1, hidden), lambda i: (0, 0)),
            pl.BlockSpec((in_features, hidden), lambda i: (0, 0)),
            pl.BlockSpec((1, in_features), lambda i: (0, 0)),
        ],
        out_specs=pl.BlockSpec((_BLK, in_features), lambda i: (i, 0)),
        out_shape=jax.ShapeDtypeStruct((n_tokens, in_features), jnp.float32),
        compiler_params=pltpu.CompilerParams(
            dimension_semantics=("parallel",)),
    )(em2d, x, W1, b1r, W2, b2r)


# speedup vs baseline: 1.2155x; 1.0143x over previous
import functools

import jax
import jax.numpy as jnp
from jax.experimental import pallas as pl
from jax.experimental.pallas import tpu as pltpu

_BLK = 1024


def _fused_mlp_kernel(em_ref, x_ref, w1_ref, b1_ref, w2_ref, b2_ref, o_ref,
                      *, hidden, num_experts):
    nh = 2
    hb = x_ref.shape[0] // nh
    for t in range(nh):
        rows = pl.ds(t * hb, hb)
        em_col = jnp.transpose(em_ref[:, rows], (1, 0))
        th = (em_col + 1) * (hidden // num_experts)
        z = jax.lax.dot_general(x_ref[rows, :], w1_ref[...],
                                (((1,), (1,)), ((), ())),
                                preferred_element_type=jnp.float32)
        z = z + b1_ref[...]
        col = jax.lax.broadcasted_iota(jnp.int32, z.shape, 1)
        h = z * z * jax.nn.sigmoid(z)
        h = jnp.where(col < th, h, 0.0)
        out = jax.lax.dot_general(h, w2_ref[...], (((1,), (1,)), ((), ())),
                                  preferred_element_type=jnp.float32)
        o_ref[rows, :] = out + b2_ref[...]


def kernel(x, expert_mask, W1, b1, W2, b2):
    n_tokens, in_features = x.shape
    hidden = W1.shape[0]
    num_experts = 8
    em2d = expert_mask.reshape(1, n_tokens)
    b1r = b1.reshape(1, hidden)
    b2r = b2.reshape(1, in_features)
    grid = (n_tokens // _BLK,)
    return pl.pallas_call(
        functools.partial(_fused_mlp_kernel, hidden=hidden,
                          num_experts=num_experts),
        grid=grid,
        in_specs=[
            pl.BlockSpec((1, _BLK), lambda i: (0, i)),
            pl.BlockSpec((_BLK, in_features), lambda i: (i, 0)),
            pl.BlockSpec((hidden, in_features), lambda i: (0, 0)),
            pl.BlockSpec((1, hidden), lambda i: (0, 0)),
            pl.BlockSpec((in_features, hidden), lambda i: (0, 0)),
            pl.BlockSpec((1, in_features), lambda i: (0, 0)),
        ],
        out_specs=pl.BlockSpec((_BLK, in_features), lambda i: (i, 0)),
        out_shape=jax.ShapeDtypeStruct((n_tokens, in_features), jnp.float32),
        compiler_params=pltpu.CompilerParams(
            dimension_semantics=("parallel",)),
    )(em2d, x, W1, b1r, W2, b2r)


# speedup vs baseline: 1.2432x; 1.0228x over previous
import functools

import jax
import jax.numpy as jnp
from jax.experimental import pallas as pl
from jax.experimental.pallas import tpu as pltpu

_BLK = 2048


def _fused_mlp_kernel(em_ref, x_ref, w1_ref, b1_ref, w2_ref, b2_ref, o_ref,
                      *, hidden, num_experts):
    nh = 4
    hb = x_ref.shape[0] // nh
    for t in range(nh):
        rows = pl.ds(t * hb, hb)
        em_col = jnp.transpose(em_ref[:, rows], (1, 0))
        th = (em_col + 1) * (hidden // num_experts)
        z = jax.lax.dot_general(x_ref[rows, :], w1_ref[...],
                                (((1,), (1,)), ((), ())),
                                preferred_element_type=jnp.float32)
        z = z + b1_ref[...]
        col = jax.lax.broadcasted_iota(jnp.int32, z.shape, 1)
        h = z * z * jax.nn.sigmoid(z)
        h = jnp.where(col < th, h, 0.0)
        out = jax.lax.dot_general(h, w2_ref[...], (((1,), (1,)), ((), ())),
                                  preferred_element_type=jnp.float32)
        o_ref[rows, :] = out + b2_ref[...]


def kernel(x, expert_mask, W1, b1, W2, b2):
    n_tokens, in_features = x.shape
    hidden = W1.shape[0]
    num_experts = 8
    em2d = expert_mask.reshape(1, n_tokens)
    b1r = b1.reshape(1, hidden)
    b2r = b2.reshape(1, in_features)
    grid = (n_tokens // _BLK,)
    return pl.pallas_call(
        functools.partial(_fused_mlp_kernel, hidden=hidden,
                          num_experts=num_experts),
        grid=grid,
        in_specs=[
            pl.BlockSpec((1, _BLK), lambda i: (0, i)),
            pl.BlockSpec((_BLK, in_features), lambda i: (i, 0)),
            pl.BlockSpec((hidden, in_features), lambda i: (0, 0)),
            pl.BlockSpec((1, hidden), lambda i: (0, 0)),
            pl.BlockSpec((in_features, hidden), lambda i: (0, 0)),
            pl.BlockSpec((1, in_features), lambda i: (0, 0)),
        ],
        out_specs=pl.BlockSpec((_BLK, in_features), lambda i: (i, 0)),
        out_shape=jax.ShapeDtypeStruct((n_tokens, in_features), jnp.float32),
        compiler_params=pltpu.CompilerParams(
            dimension_semantics=("parallel",)),
    )(em2d, x, W1, b1r, W2, b2r)


# speedup vs baseline: 1.2684x; 1.0203x over previous
import functools

import jax
import jax.numpy as jnp
from jax.experimental import pallas as pl
from jax.experimental.pallas import tpu as pltpu

_BLK = 2048


def _fused_mlp_kernel(em_ref, x_ref, w1_ref, b1_ref, w2_ref, b2_ref, o_ref,
                      *, hidden, num_experts):
    nh = 4
    hb = x_ref.shape[0] // nh
    for t in range(nh):
        rows = pl.ds(t * hb, hb)
        em_col = jnp.transpose(em_ref[:, rows], (1, 0))
        th = (em_col + 1) * (hidden // num_experts)
        z = jax.lax.dot_general(x_ref[rows, :], w1_ref[...],
                                (((1,), (1,)), ((), ())),
                                preferred_element_type=jnp.float32)
        z = z + b1_ref[...]
        col = jax.lax.broadcasted_iota(jnp.int32, z.shape, 1)
        h = (0.5 * z * z) * (1.0 + jnp.tanh(0.5 * z))
        h = jnp.where(col < th, h, 0.0)
        out = jax.lax.dot_general(h, w2_ref[...], (((1,), (1,)), ((), ())),
                                  preferred_element_type=jnp.float32)
        o_ref[rows, :] = out + b2_ref[...]


def kernel(x, expert_mask, W1, b1, W2, b2):
    n_tokens, in_features = x.shape
    hidden = W1.shape[0]
    num_experts = 8
    em2d = expert_mask.reshape(1, n_tokens)
    b1r = b1.reshape(1, hidden)
    b2r = b2.reshape(1, in_features)
    grid = (n_tokens // _BLK,)
    return pl.pallas_call(
        functools.partial(_fused_mlp_kernel, hidden=hidden,
                          num_experts=num_experts),
        grid=grid,
        in_specs=[
            pl.BlockSpec((1, _BLK), lambda i: (0, i)),
            pl.BlockSpec((_BLK, in_features), lambda i: (i, 0)),
            pl.BlockSpec((hidden, in_features), lambda i: (0, 0)),
            pl.BlockSpec((1, hidden), lambda i: (0, 0)),
            pl.BlockSpec((in_features, hidden), lambda i: (0, 0)),
            pl.BlockSpec((1, in_features), lambda i: (0, 0)),
        ],
        out_specs=pl.BlockSpec((_BLK, in_features), lambda i: (i, 0)),
        out_shape=jax.ShapeDtypeStruct((n_tokens, in_features), jnp.float32),
        compiler_params=pltpu.CompilerParams(
            dimension_semantics=("parallel",)),
    )(em2d, x, W1, b1r, W2, b2r)
